# baseline (speedup 1.0000x reference)
import functools

import jax
import jax.numpy as jnp
from jax import lax
from jax.experimental import pallas as pl
from jax.experimental.pallas import tpu as pltpu
from jax.experimental.pallas import tpu_sc as plsc

_NC = 2
_NS = 16
_EB = 128
_AW = 128


def _segsum_kernel(R, N, KB):
  NA = -(-N // 128) * 128
  RPS = NA // _NS
  DEP = 2
  SB = max(R - 1, 1).bit_length()
  assert KB % DEP == 0 and KB >= DEP
  assert SB + (NA - 1).bit_length() <= 31

  mesh = plsc.VectorSubcoreMesh(core_axis_name="c", subcore_axis_name="s",
                                num_cores=_NC, num_subcores=_NS)

  @functools.partial(
      pl.kernel,
      out_type=jax.ShapeDtypeStruct((_NC, NA, _AW), jnp.float32),
      mesh=mesh,
      scratch_types=[
          pltpu.VMEM((KB * _EB,), jnp.int32),
          pltpu.VMEM((DEP, _EB), jnp.int32),
          pltpu.VMEM((_EB,), jnp.int32),
          pltpu.VMEM((DEP, _EB, _AW), jnp.float32),
          pltpu.VMEM_SHARED((NA, _AW), jnp.float32),
          pltpu.SemaphoreType.DMA((DEP,)),
      ],
  )
  def seg(rows_hbm, pk_hbm, zero_hbm, out_hbm,
          pk_v, src_v, dst_v, bufv, acc, sems):
    bufs = [bufv.at[b] for b in range(DEP)]
    sem_l = [sems.at[b] for b in range(DEP)]
    c = lax.axis_index("c")
    t = lax.axis_index("s")
    pltpu.sync_copy(pk_hbm.at[c, t], pk_v)
    pltpu.sync_copy(zero_hbm.at[pl.ds(t * RPS, RPS)],
                    acc.at[pl.ds(t * RPS, RPS)])

    def unpack_src(j, b):
      for k in range(_EB // 16):
        v = pk_v[pl.ds(j * _EB + 16 * k, 16)]
        src_v[b, pl.ds(16 * k, 16)] = v & ((1 << SB) - 1)

    def unpack_dst(j):
      for k in range(_EB // 16):
        v = pk_v[pl.ds(j * _EB + 16 * k, 16)]
        dst_v[pl.ds(16 * k, 16)] = lax.shift_right_logical(v, SB)

    plsc.subcore_barrier()

    for b in range(DEP):
      unpack_src(jnp.int32(b), b)
      pltpu.async_copy(rows_hbm.at[src_v.at[b]], bufs[b], sem_l[b])

    def body(i, carry):
      for b in range(DEP):
        j = DEP * i + b
        pltpu.make_async_copy(rows_hbm.at[src_v.at[b]], bufs[b],
                              sem_l[b]).wait()
        unpack_dst(j)
        pltpu.sync_copy(bufs[b], acc.at[dst_v], add=True)
        jc = jnp.minimum(j + DEP, KB - 1)
        unpack_src(jc, b)
        pltpu.async_copy(rows_hbm.at[src_v.at[b]], bufs[b], sem_l[b])
      return carry

    lax.fori_loop(0, KB // DEP, body, 0, unroll=False)
    for b in range(DEP):
      pltpu.make_async_copy(rows_hbm.at[src_v.at[b]], bufs[b],
                            sem_l[b]).wait()
    plsc.subcore_barrier()
    pltpu.sync_copy(acc.at[pl.ds(t * RPS, RPS)],
                    out_hbm.at[c, pl.ds(t * RPS, RPS)])

  return seg


def kernel(X, edge_index, eps_noise, param, Wu, bu, eps_u,
           Wmu, bmu, eps_mu, Wsig, bsig, eps_sig, rk_logit):
  f32 = jnp.float32
  S, N, NOISE = eps_noise.shape
  D = X.shape[2]
  E = edge_index.shape[1]
  DU = Wu.shape[1]
  DZ = Wmu.shape[1]
  J = param.shape[0]
  K = S - J

  Xr = X[0]
  Wux, Wun = Wu[:D], Wu[D:]
  Wmx, Wmh = Wmu[:D], Wmu[D:]
  Wsx, Wsh = Wsig[:D], Wsig[D:]

  src = edge_index[0]
  dst = edge_index[1]
  NA = -(-N // 128) * 128
  zero_na = jnp.zeros((NA, _AW), f32)
  zpad = jnp.zeros((8, _AW), f32)

  BN = 1000
  GN = N // BN

  NZ2 = jnp.concatenate([eps_noise[0], eps_noise[1]], axis=1)
  T1 = jnp.concatenate([Xr, NZ2, zpad], axis=0)
  KB1 = -(-(-(-E // (_NS * _EB))) // 2) * 2
  per1 = _NS * KB1 * _EB
  SB1 = max(2 * N + 7, 1).bit_length()
  s1 = jnp.concatenate([src, jnp.full((per1 - E,), 2 * N, jnp.int32)])
  s1 = jnp.concatenate([s1, jnp.where(s1 < N, s1 + N, s1)])
  d1 = jnp.concatenate([dst, jnp.zeros((per1 - E,), jnp.int32)])
  d1 = jnp.concatenate([d1, d1])
  pk1 = ((d1 << SB1) | s1).reshape(2, _NS, KB1 * _EB)

  seg1 = _segsum_kernel(2 * N + 8, N, KB1)
  Agg1 = seg1(T1, pk1, zero_na)

  eps3 = jnp.stack([eps_u, eps_mu, eps_sig]).reshape(3, 1)
  bu2 = jnp.reshape(bu, (1, DU))
  bmg = jnp.concatenate([bmu, bsig]).reshape(1, 2 * DZ)

  def sA_body(eps_ref, x_ref, ax_ref, anz_ref, nz_ref, wux_ref, wun_ref,
              wmx_ref, wsx_ref, bu_ref, bmg_ref, h2_ref, pmg_ref):
    eu = 1.0 + eps_ref[0, 0]
    em = 1.0 + eps_ref[1, 0]
    es = 1.0 + eps_ref[2, 0]
    x = x_ref[...]
    ax = ax_ref[0]
    t1u = eu * x + ax
    au = jnp.dot(t1u, wux_ref[...], preferred_element_type=f32)
    hs = []
    for s in range(S):
      t1n = eu * nz_ref[s] + anz_ref[0][:, s * NOISE:(s + 1) * NOISE]
      h = jnp.maximum(au + jnp.dot(t1n, wun_ref[...],
                                   preferred_element_type=f32) + bu_ref[...],
                      0.0)
      hs.append(h)
    hs.append(jnp.zeros((x.shape[0], _AW - S * DU), f32))
    h2_ref[...] = jnp.concatenate(hs, axis=1)
    pm = jnp.dot(em * x + ax, wmx_ref[...], preferred_element_type=f32)
    pg = jnp.dot(es * x + ax, wsx_ref[...], preferred_element_type=f32)
    pmg_ref[...] = jnp.concatenate([pm, pg], axis=1) + bmg_ref[...]

  H2, PMG = pl.pallas_call(
      sA_body,
      grid=(GN,),
      in_specs=[
          pl.BlockSpec(memory_space=pltpu.SMEM),
          pl.BlockSpec((BN, D), lambda i: (i, 0)),
          pl.BlockSpec((1, BN, _AW), lambda i: (0, i, 0)),
          pl.BlockSpec((1, BN, _AW), lambda i: (1, i, 0)),
          pl.BlockSpec((S, BN, NOISE), lambda i: (0, i, 0)),
          pl.BlockSpec((D, DU), lambda i: (0, 0)),
          pl.BlockSpec((NOISE, DU), lambda i: (0, 0)),
          pl.BlockSpec((D, DZ), lambda i: (0, 0)),
          pl.BlockSpec((D, DZ), lambda i: (0, 0)),
          pl.BlockSpec((1, DU), lambda i: (0, 0)),
          pl.BlockSpec((1, 2 * DZ), lambda i: (0, 0)),
      ],
      out_specs=[
          pl.BlockSpec((BN, _AW), lambda i: (i, 0)),
          pl.BlockSpec((BN, 2 * DZ), lambda i: (i, 0)),
      ],
      out_shape=[
          jax.ShapeDtypeStruct((N, _AW), f32),
          jax.ShapeDtypeStruct((N, 2 * DZ), f32),
      ],
  )(eps3, Xr, Agg1, Agg1, eps_noise, Wux, Wun, Wmx, Wsx, bu2, bmg)

  T2 = jnp.concatenate([H2, zpad], axis=0)
  KB2 = -(-(-(-E // (2 * _NS * _EB))) // 2) * 2
  SB2 = max(N + 7, 1).bit_length()
  total2 = 2 * _NS * KB2 * _EB
  s2 = jnp.concatenate([src, jnp.full((total2 - E,), N, jnp.int32)])
  d2 = jnp.concatenate([dst, jnp.zeros((total2 - E,), jnp.int32)])
  pk2 = ((d2 << SB2) | s2).reshape(2, _NS, KB2 * _EB)
  seg2 = _segsum_kernel(N + 8, N, KB2)
  Agg2 = seg2(T2, pk2, zero_na)

  rkl2 = jnp.reshape(rk_logit, (1, DZ))

  def sB_body(eps_ref, h2_ref, a2a_ref, a2b_ref, pmg_ref, wmh_ref, wsh_ref,
              par_ref, rkl_ref, mu_ref, sg_ref, z_ref, zr_ref):
    em = 1.0 + eps_ref[1, 0]
    es = 1.0 + eps_ref[2, 0]
    ah_full = a2a_ref[0] + a2b_ref[0]
    rk = jax.nn.sigmoid(rkl_ref[...])
    pmg = pmg_ref[...]
    for s in range(S):
      h = h2_ref[:, s * DU:(s + 1) * DU]
      ah = ah_full[:, s * DU:(s + 1) * DU]
      m = pmg[:, :DZ] + jnp.dot(em * h + ah, wmh_ref[...],
                                preferred_element_type=f32)
      g = pmg[:, DZ:] + jnp.dot(es * h + ah, wsh_ref[...],
                                preferred_element_type=f32)
      sg = jnp.exp(g * 0.5)
      mu_ref[s] = m
      sg_ref[s] = sg
      if s >= K:
        j = s - K
        z = par_ref[j] * sg + m
        z_ref[j] = z
        zr_ref[j] = z * rk

  mu, sigma, Z, ZR = pl.pallas_call(
      sB_body,
      grid=(GN,),
      in_specs=[
          pl.BlockSpec(memory_space=pltpu.SMEM),
          pl.BlockSpec((BN, _AW), lambda i: (i, 0)),
          pl.BlockSpec((1, BN, _AW), lambda i: (0, i, 0)),
          pl.BlockSpec((1, BN, _AW), lambda i: (1, i, 0)),
          pl.BlockSpec((BN, 2 * DZ), lambda i: (i, 0)),
          pl.BlockSpec((DU, DZ), lambda i: (0, 0)),
          pl.BlockSpec((DU, DZ), lambda i: (0, 0)),
          pl.BlockSpec((J, BN, DZ), lambda i: (0, i, 0)),
          pl.BlockSpec((1, DZ), lambda i: (0, 0)),
      ],
      out_specs=[
          pl.BlockSpec((S, BN, DZ), lambda i: (0, i, 0)),
          pl.BlockSpec((S, BN, DZ), lambda i: (0, i, 0)),
          pl.BlockSpec((J, BN, DZ), lambda i: (0, i, 0)),
          pl.BlockSpec((J, BN, DZ), lambda i: (0, i, 0)),
      ],
      out_shape=[
          jax.ShapeDtypeStruct((S, N, DZ), f32),
          jax.ShapeDtypeStruct((S, N, DZ), f32),
          jax.ShapeDtypeStruct((J, N, DZ), f32),
          jax.ShapeDtypeStruct((J, N, DZ), f32),
      ],
  )(eps3, H2, Agg2, Agg2, PMG, Wmh, Wsh, param, rkl2)

  TM = 200
  GM = N // TM

  def dec_body(z_ref, zr_ref, out_ref):
    l = lax.dot_general(z_ref[0], zr_ref[0], (((1,), (1,)), ((), ())),
                        preferred_element_type=f32)
    l = jnp.minimum(l, 10.0)
    out_ref[0] = 1.0 - jnp.exp(-jnp.exp(l))

  adj = pl.pallas_call(
      dec_body,
      grid=(J, GM),
      in_specs=[
          pl.BlockSpec((1, TM, DZ), lambda a, i: (a, i, 0)),
          pl.BlockSpec((1, N, DZ), lambda a, i: (a, 0, 0)),
      ],
      out_specs=pl.BlockSpec((1, TM, N), lambda a, i: (a, i, 0)),
      out_shape=jax.ShapeDtypeStruct((J, N, N), f32),
      compiler_params=pltpu.CompilerParams(
          dimension_semantics=("parallel", "parallel")),
  )(Z, ZR)

  return (adj, mu, sigma, Z)


# speedup vs baseline: 1.2134x; 1.2134x over previous
import functools

import jax
import jax.numpy as jnp
from jax import lax
from jax.experimental import pallas as pl
from jax.experimental.pallas import tpu as pltpu
from jax.experimental.pallas import tpu_sc as plsc

_NC = 2
_NS = 16
_EB = 128
_AW = 128


def _segsum_kernel(R, N, KB, DEP=1):
  NA = -(-N // 128) * 128
  RPS = NA // _NS

  mesh = plsc.VectorSubcoreMesh(core_axis_name="c", subcore_axis_name="s",
                                num_cores=_NC, num_subcores=_NS)

  @functools.partial(
      pl.kernel,
      out_type=jax.ShapeDtypeStruct((_NC, NA, _AW), jnp.float32),
      mesh=mesh,
      scratch_types=[
          pltpu.VMEM((KB, _EB), jnp.int32),
          pltpu.VMEM((KB, _EB), jnp.int32),
          pltpu.VMEM((DEP, _EB, _AW), jnp.float32),
          pltpu.VMEM_SHARED((NA, _AW), jnp.float32),
          pltpu.SemaphoreType.DMA((DEP,)),
      ],
  )
  def seg(rows_hbm, src_hbm, dst_hbm, zero_hbm, out_hbm,
          src_v, dst_v, bufv, acc, sems):
    c = lax.axis_index("c")
    t = lax.axis_index("s")
    pltpu.sync_copy(src_hbm.at[c, t], src_v)
    pltpu.sync_copy(dst_hbm.at[c, t], dst_v)
    pltpu.sync_copy(zero_hbm.at[pl.ds(t * RPS, RPS)],
                    acc.at[pl.ds(t * RPS, RPS)])
    plsc.subcore_barrier()

    if DEP == 1:
      def body(j, carry):
        pltpu.async_copy(rows_hbm.at[src_v.at[j]], bufv.at[0],
                         sems.at[0]).wait()
        pltpu.sync_copy(bufv.at[0], acc.at[dst_v.at[j]], add=True)
        return carry

      lax.fori_loop(0, KB, body, 0, unroll=False)
    else:
      for b in range(DEP):
        pltpu.async_copy(rows_hbm.at[src_v.at[b]], bufv.at[b], sems.at[b])

      def body(i, carry):
        for b in range(DEP):
          j = DEP * i + b
          pltpu.make_async_copy(rows_hbm.at[src_v.at[j]], bufv.at[b],
                                sems.at[b]).wait()
          pltpu.sync_copy(bufv.at[b], acc.at[dst_v.at[j]], add=True)
          jc = jnp.minimum(j + DEP, KB - 1)
          pltpu.async_copy(rows_hbm.at[src_v.at[jc]], bufv.at[b],
                           sems.at[b])
        return carry

      lax.fori_loop(0, KB // DEP, body, 0, unroll=False)
      for b in range(DEP):
        pltpu.make_async_copy(rows_hbm.at[src_v.at[0]], bufv.at[b],
                              sems.at[b]).wait()
    plsc.subcore_barrier()
    pltpu.sync_copy(acc.at[pl.ds(t * RPS, RPS)],
                    out_hbm.at[c, pl.ds(t * RPS, RPS)])

  return seg


def kernel(X, edge_index, eps_noise, param, Wu, bu, eps_u,
           Wmu, bmu, eps_mu, Wsig, bsig, eps_sig, rk_logit):
  f32 = jnp.float32
  S, N, NOISE = eps_noise.shape
  D = X.shape[2]
  E = edge_index.shape[1]
  DU = Wu.shape[1]
  DZ = Wmu.shape[1]
  J = param.shape[0]
  K = S - J

  Xr = X[0]
  Wux, Wun = Wu[:D], Wu[D:]
  Wmx, Wmh = Wmu[:D], Wmu[D:]
  Wsx, Wsh = Wsig[:D], Wsig[D:]

  src = edge_index[0]
  dst = edge_index[1]
  NA = -(-N // 128) * 128
  zero_na = jnp.zeros((NA, _AW), f32)
  zpad = jnp.zeros((8, _AW), f32)

  BN = 1000
  GN = N // BN

  NZ2 = jnp.concatenate([eps_noise[0], eps_noise[1]], axis=1)
  T1 = jnp.concatenate([Xr, NZ2, zpad], axis=0)
  KB1 = -(-E // (_NS * _EB))
  per1 = _NS * KB1 * _EB
  s1 = jnp.concatenate([src, jnp.full((per1 - E,), 2 * N, jnp.int32)])
  s1 = jnp.concatenate([s1, jnp.where(s1 < N, s1 + N, s1)])
  src1 = s1.reshape(2, _NS, KB1, _EB)
  d1 = jnp.concatenate([dst, jnp.zeros((per1 - E,), jnp.int32)])
  dst1 = jnp.concatenate([d1, d1]).reshape(2, _NS, KB1, _EB)

  seg1 = _segsum_kernel(2 * N + 8, N, KB1)
  Agg1 = seg1(T1, src1, dst1, zero_na)

  eps3 = jnp.stack([eps_u, eps_mu, eps_sig]).reshape(3, 1)
  bu2 = jnp.reshape(bu, (1, DU))
  bmg = jnp.concatenate([bmu, bsig]).reshape(1, 2 * DZ)

  def sA_body(eps_ref, x_ref, ax_ref, anz_ref, nz_ref, wux_ref, wun_ref,
              wmx_ref, wsx_ref, bu_ref, bmg_ref, h2_ref, pmg_ref):
    eu = 1.0 + eps_ref[0, 0]
    em = 1.0 + eps_ref[1, 0]
    es = 1.0 + eps_ref[2, 0]
    x = x_ref[...]
    ax = ax_ref[0]
    t1u = eu * x + ax
    au = jnp.dot(t1u, wux_ref[...], preferred_element_type=f32)
    hs = []
    for s in range(S):
      t1n = eu * nz_ref[s] + anz_ref[0][:, s * NOISE:(s + 1) * NOISE]
      h = jnp.maximum(au + jnp.dot(t1n, wun_ref[...],
                                   preferred_element_type=f32) + bu_ref[...],
                      0.0)
      hs.append(h)
    hs.append(jnp.zeros((x.shape[0], _AW - S * DU), f32))
    h2_ref[...] = jnp.concatenate(hs, axis=1)
    pm = jnp.dot(em * x + ax, wmx_ref[...], preferred_element_type=f32)
    pg = jnp.dot(es * x + ax, wsx_ref[...], preferred_element_type=f32)
    pmg_ref[...] = jnp.concatenate([pm, pg], axis=1) + bmg_ref[...]

  H2, PMG = pl.pallas_call(
      sA_body,
      grid=(GN,),
      in_specs=[
          pl.BlockSpec(memory_space=pltpu.SMEM),
          pl.BlockSpec((BN, D), lambda i: (i, 0)),
          pl.BlockSpec((1, BN, _AW), lambda i: (0, i, 0)),
          pl.BlockSpec((1, BN, _AW), lambda i: (1, i, 0)),
          pl.BlockSpec((S, BN, NOISE), lambda i: (0, i, 0)),
          pl.BlockSpec((D, DU), lambda i: (0, 0)),
          pl.BlockSpec((NOISE, DU), lambda i: (0, 0)),
          pl.BlockSpec((D, DZ), lambda i: (0, 0)),
          pl.BlockSpec((D, DZ), lambda i: (0, 0)),
          pl.BlockSpec((1, DU), lambda i: (0, 0)),
          pl.BlockSpec((1, 2 * DZ), lambda i: (0, 0)),
      ],
      out_specs=[
          pl.BlockSpec((BN, _AW), lambda i: (i, 0)),
          pl.BlockSpec((BN, 2 * DZ), lambda i: (i, 0)),
      ],
      out_shape=[
          jax.ShapeDtypeStruct((N, _AW), f32),
          jax.ShapeDtypeStruct((N, 2 * DZ), f32),
      ],
  )(eps3, Xr, Agg1, Agg1, eps_noise, Wux, Wun, Wmx, Wsx, bu2, bmg)

  T2 = jnp.concatenate([H2, zpad], axis=0)
  KB2 = -(-(-(-E // (2 * _NS * _EB))) // 2) * 2
  total2 = 2 * _NS * KB2 * _EB
  s2 = jnp.concatenate([src, jnp.full((total2 - E,), N, jnp.int32)])
  d2 = jnp.concatenate([dst, jnp.zeros((total2 - E,), jnp.int32)])
  src2 = s2.reshape(2, _NS, KB2, _EB)
  dst2 = d2.reshape(2, _NS, KB2, _EB)
  seg2 = _segsum_kernel(N + 8, N, KB2, DEP=2)
  Agg2 = seg2(T2, src2, dst2, zero_na)

  rkl2 = jnp.reshape(rk_logit, (1, DZ))

  def sB_body(eps_ref, h2_ref, a2a_ref, a2b_ref, pmg_ref, wmh_ref, wsh_ref,
              par_ref, rkl_ref, mu_ref, sg_ref, z_ref, zr_ref):
    em = 1.0 + eps_ref[1, 0]
    es = 1.0 + eps_ref[2, 0]
    ah_full = a2a_ref[0] + a2b_ref[0]
    rk = jax.nn.sigmoid(rkl_ref[...])
    pmg = pmg_ref[...]
    for s in range(S):
      h = h2_ref[:, s * DU:(s + 1) * DU]
      ah = ah_full[:, s * DU:(s + 1) * DU]
      m = pmg[:, :DZ] + jnp.dot(em * h + ah, wmh_ref[...],
                                preferred_element_type=f32)
      g = pmg[:, DZ:] + jnp.dot(es * h + ah, wsh_ref[...],
                                preferred_element_type=f32)
      sg = jnp.exp(g * 0.5)
      mu_ref[s] = m
      sg_ref[s] = sg
      if s >= K:
        j = s - K
        z = par_ref[j] * sg + m
        z_ref[j] = z
        zr_ref[j] = z * rk

  mu, sigma, Z, ZR = pl.pallas_call(
      sB_body,
      grid=(GN,),
      in_specs=[
          pl.BlockSpec(memory_space=pltpu.SMEM),
          pl.BlockSpec((BN, _AW), lambda i: (i, 0)),
          pl.BlockSpec((1, BN, _AW), lambda i: (0, i, 0)),
          pl.BlockSpec((1, BN, _AW), lambda i: (1, i, 0)),
          pl.BlockSpec((BN, 2 * DZ), lambda i: (i, 0)),
          pl.BlockSpec((DU, DZ), lambda i: (0, 0)),
          pl.BlockSpec((DU, DZ), lambda i: (0, 0)),
          pl.BlockSpec((J, BN, DZ), lambda i: (0, i, 0)),
          pl.BlockSpec((1, DZ), lambda i: (0, 0)),
      ],
      out_specs=[
          pl.BlockSpec((S, BN, DZ), lambda i: (0, i, 0)),
          pl.BlockSpec((S, BN, DZ), lambda i: (0, i, 0)),
          pl.BlockSpec((J, BN, DZ), lambda i: (0, i, 0)),
          pl.BlockSpec((J, BN, DZ), lambda i: (0, i, 0)),
      ],
      out_shape=[
          jax.ShapeDtypeStruct((S, N, DZ), f32),
          jax.ShapeDtypeStruct((S, N, DZ), f32),
          jax.ShapeDtypeStruct((J, N, DZ), f32),
          jax.ShapeDtypeStruct((J, N, DZ), f32),
      ],
  )(eps3, H2, Agg2, Agg2, PMG, Wmh, Wsh, param, rkl2)

  TM = 400
  GM = N // TM

  def dec_body(z_ref, zr_ref, out_ref):
    l = lax.dot_general(z_ref[0], zr_ref[0], (((1,), (1,)), ((), ())),
                        preferred_element_type=f32)
    l = jnp.minimum(l, 10.0)
    out_ref[0] = 1.0 - jnp.exp(-jnp.exp(l))

  adj = pl.pallas_call(
      dec_body,
      grid=(J, GM),
      in_specs=[
          pl.BlockSpec((1, TM, DZ), lambda a, i: (a, i, 0)),
          pl.BlockSpec((1, N, DZ), lambda a, i: (a, 0, 0)),
      ],
      out_specs=pl.BlockSpec((1, TM, N), lambda a, i: (a, i, 0)),
      out_shape=jax.ShapeDtypeStruct((J, N, N), f32),
      compiler_params=pltpu.CompilerParams(
          dimension_semantics=("parallel", "parallel")),
  )(Z, ZR)

  return (adj, mu, sigma, Z)
